# Initial kernel scaffold; baseline (speedup 1.0000x reference)
import jax
import jax.numpy as jnp
from jax.experimental import pallas as pl


def kernel(node_feats, edge_index, edge_feats, W1, b1, W2, b2):
    raise NotImplementedError("write your pallas kernel here")


# speedup vs baseline: 4.5820x; 4.5820x over previous
import functools

import jax
import jax.numpy as jnp
from jax import lax
from jax.experimental import pallas as pl
from jax.experimental.pallas import tpu as pltpu
from jax.experimental.pallas import tpu_sc as plsc

NODE_DIM = 128
EDGE_DIM = 16
N_NODES = 10000
N_EDGES = 320000

_HI = jax.lax.Precision.HIGHEST


def _proj_body(x_ref, wa_ref, wb_ref, p_ref, q_ref):
    x = x_ref[...]
    p_ref[...] = jnp.dot(x, wa_ref[...], precision=_HI,
                         preferred_element_type=jnp.float32)
    q_ref[...] = jnp.dot(x, wb_ref[...], precision=_HI,
                         preferred_element_type=jnp.float32)


def _project(node_feats, W1a, W1b):
    return pl.pallas_call(
        _proj_body,
        out_shape=[jax.ShapeDtypeStruct((N_NODES, EDGE_DIM), jnp.float32)] * 2,
    )(node_feats, W1a, W1b)


def _sc_gather(P, Q, src, dst):
    info = plsc.get_sparse_core_info()
    NC, NS = info.num_cores, info.num_subcores
    NW = NC * NS
    EPW = N_EDGES // NW
    CH = 2000
    NCH = EPW // CH

    mesh = plsc.VectorSubcoreMesh(core_axis_name="c", subcore_axis_name="s")

    @functools.partial(
        pl.kernel,
        mesh=mesh,
        out_type=(jax.ShapeDtypeStruct((N_EDGES, EDGE_DIM), jnp.float32),) * 2,
        scratch_types=[
            pltpu.VMEM((CH,), jnp.int32),
            pltpu.VMEM((CH,), jnp.int32),
            pltpu.VMEM((CH, EDGE_DIM), jnp.float32),
            pltpu.VMEM((CH, EDGE_DIM), jnp.float32),
            pltpu.SemaphoreType.DMA,
            pltpu.SemaphoreType.DMA,
        ],
        compiler_params=pltpu.CompilerParams(use_tc_tiling_on_sc=False),
    )
    def body(p_hbm, q_hbm, src_hbm, dst_hbm, g1_hbm, g2_hbm,
             src_v, dst_v, r1, r2, s1, s2):
        wid = lax.axis_index("s") * NC + lax.axis_index("c")
        base = wid * EPW

        def chunk(k, carry):
            off = pl.multiple_of(base + k * CH, 8)
            pltpu.sync_copy(src_hbm.at[pl.ds(off, CH)], src_v)
            pltpu.sync_copy(dst_hbm.at[pl.ds(off, CH)], dst_v)
            c1 = pltpu.async_copy(p_hbm.at[src_v], r1, s1)
            c2 = pltpu.async_copy(q_hbm.at[dst_v], r2, s2)
            c1.wait()
            c2.wait()
            pltpu.sync_copy(r1, g1_hbm.at[pl.ds(off, CH)])
            pltpu.sync_copy(r2, g2_hbm.at[pl.ds(off, CH)])
            return carry

        lax.fori_loop(0, NCH, chunk, 0)

    return body(P, Q, src, dst)


def _edge_body(g1_ref, g2_ref, e_ref, w1_ref, b1_ref, w2_ref, b2_ref, o_ref):
    h = (g1_ref[...] + g2_ref[...]
         + jnp.dot(e_ref[...], w1_ref[...], precision=_HI,
                   preferred_element_type=jnp.float32)
         + b1_ref[...])
    h = jnp.where(h >= 0, h, 0.2 * h)
    o_ref[...] = (jnp.dot(h, w2_ref[...], precision=_HI,
                          preferred_element_type=jnp.float32)
                  + b2_ref[...])


def _edge_mlp(g1v, g2v, ev, w1e_big, b1_big, w2_big, b2_big):
    rows = N_EDGES * EDGE_DIM // 128
    BLK = 4000
    grid = rows // BLK
    full = lambda i: (0, 0)
    blk = lambda i: (i, 0)
    return pl.pallas_call(
        _edge_body,
        grid=(grid,),
        in_specs=[
            pl.BlockSpec((BLK, 128), blk),
            pl.BlockSpec((BLK, 128), blk),
            pl.BlockSpec((BLK, 128), blk),
            pl.BlockSpec((128, 128), full),
            pl.BlockSpec((1, 128), full),
            pl.BlockSpec((128, 128), full),
            pl.BlockSpec((1, 128), full),
        ],
        out_specs=pl.BlockSpec((BLK, 128), blk),
        out_shape=jax.ShapeDtypeStruct((rows, 128), jnp.float32),
    )(g1v, g2v, ev, w1e_big, b1_big, w2_big, b2_big)


def kernel(node_feats, edge_index, edge_feats, W1, b1, W2, b2):
    src = edge_index[0].astype(jnp.int32)
    dst = edge_index[1].astype(jnp.int32)
    W1a = W1[:NODE_DIM]
    W1b = W1[NODE_DIM:2 * NODE_DIM]
    W1e = W1[2 * NODE_DIM:]

    P, Q = _project(node_feats, W1a, W1b)
    G1, G2 = _sc_gather(P, Q, src, dst)

    eye8 = jnp.eye(8, dtype=jnp.float32)
    w1e_big = jnp.kron(eye8, W1e)
    w2_big = jnp.kron(eye8, W2)
    b1_big = jnp.tile(b1, 8)[None, :]
    b2_big = jnp.tile(b2, 8)[None, :]

    rows = N_EDGES * EDGE_DIM // 128
    out = _edge_mlp(
        G1.reshape(rows, 128), G2.reshape(rows, 128),
        edge_feats.reshape(rows, 128),
        w1e_big, b1_big, w2_big, b2_big,
    )
    return out.reshape(N_EDGES, EDGE_DIM)


# speedup vs baseline: 4.6518x; 1.0152x over previous
import functools

import jax
import jax.numpy as jnp
from jax import lax
from jax.experimental import pallas as pl
from jax.experimental.pallas import tpu as pltpu
from jax.experimental.pallas import tpu_sc as plsc

NODE_DIM = 128
EDGE_DIM = 16
N_NODES = 10000
N_NODES_PAD = 10240
N_EDGES = 320000

_HI = jax.lax.Precision.HIGHEST


def _proj_body(x_ref, wa_ref, wb_ref, p_ref, q_ref):
    x = x_ref[...]
    p_ref[...] = jnp.dot(x, wa_ref[...], precision=_HI,
                         preferred_element_type=jnp.float32)
    q_ref[...] = jnp.dot(x, wb_ref[...], precision=_HI,
                         preferred_element_type=jnp.float32)


def _project(node_feats, W1a, W1b):
    return pl.pallas_call(
        _proj_body,
        out_shape=[jax.ShapeDtypeStruct((N_NODES, EDGE_DIM),
                                        jnp.float32)] * 2,
    )(node_feats, W1a, W1b)


def _sc_gather(P, Q, src, dst):
    info = plsc.get_sparse_core_info()
    NC, NS = info.num_cores, info.num_subcores
    NW = NC * NS
    EPW = N_EDGES // NW
    CH = 2000
    NCH = EPW // CH

    mesh = plsc.VectorSubcoreMesh(core_axis_name="c", subcore_axis_name="s")

    @functools.partial(
        pl.kernel,
        mesh=mesh,
        out_type=(jax.ShapeDtypeStruct((N_EDGES, EDGE_DIM), jnp.float32),) * 2,
        scratch_types=[
            pltpu.VMEM((CH,), jnp.int32),
            pltpu.VMEM((CH,), jnp.int32),
            pltpu.VMEM((CH, EDGE_DIM), jnp.float32),
            pltpu.VMEM((CH, EDGE_DIM), jnp.float32),
            pltpu.SemaphoreType.DMA,
            pltpu.SemaphoreType.DMA,
        ],
        compiler_params=pltpu.CompilerParams(use_tc_tiling_on_sc=False),
    )
    def body(p_hbm, q_hbm, src_hbm, dst_hbm, g1_hbm, g2_hbm,
             src_v, dst_v, r1, r2, s1, s2):
        wid = lax.axis_index("s") * NC + lax.axis_index("c")
        base = wid * EPW

        def chunk(k, carry):
            off = pl.multiple_of(base + k * CH, 8)
            pltpu.sync_copy(src_hbm.at[pl.ds(off, CH)], src_v)
            pltpu.sync_copy(dst_hbm.at[pl.ds(off, CH)], dst_v)
            c1 = pltpu.async_copy(p_hbm.at[src_v], r1, s1)
            c2 = pltpu.async_copy(q_hbm.at[dst_v], r2, s2)
            c1.wait()
            c2.wait()
            pltpu.sync_copy(r1, g1_hbm.at[pl.ds(off, CH)])
            pltpu.sync_copy(r2, g2_hbm.at[pl.ds(off, CH)])
            return carry

        lax.fori_loop(0, NCH, chunk, 0)

    return body(P, Q, src, dst)


_BLK = 4000


def _edge_body(g1_ref, g2_ref, et_ref, w1_ref, b1_ref, w2_ref, b2_ref,
               ot_ref):
    et = et_ref[...]
    e = jnp.concatenate(
        [et[:, g * _BLK:(g + 1) * _BLK].T for g in range(8)], axis=1)
    h = (g1_ref[...] + g2_ref[...]
         + jnp.dot(e, w1_ref[...], precision=_HI,
                   preferred_element_type=jnp.float32)
         + b1_ref[...])
    h = jnp.where(h >= 0, h, 0.2 * h)
    o = (jnp.dot(h, w2_ref[...], precision=_HI,
                 preferred_element_type=jnp.float32)
         + b2_ref[...])
    ot_ref[...] = jnp.concatenate(
        [o[:, g * EDGE_DIM:(g + 1) * EDGE_DIM].T for g in range(8)], axis=1)


def _edge_mlp(g1v, g2v, et, w1e_big, b1_big, w2_big, b2_big):
    rows = N_EDGES * EDGE_DIM // 128
    grid = rows // _BLK
    full = lambda i: (0, 0)
    blk = lambda i: (i, 0)
    lane_blk = lambda i: (0, i)
    return pl.pallas_call(
        _edge_body,
        grid=(grid,),
        in_specs=[
            pl.BlockSpec((_BLK, 128), blk),
            pl.BlockSpec((_BLK, 128), blk),
            pl.BlockSpec((EDGE_DIM, _BLK * 8), lane_blk),
            pl.BlockSpec((128, 128), full),
            pl.BlockSpec((1, 128), full),
            pl.BlockSpec((128, 128), full),
            pl.BlockSpec((1, 128), full),
        ],
        out_specs=pl.BlockSpec((EDGE_DIM, _BLK * 8), lane_blk),
        out_shape=jax.ShapeDtypeStruct((EDGE_DIM, N_EDGES), jnp.float32),
    )(g1v, g2v, et, w1e_big, b1_big, w2_big, b2_big)


def _permute(idx):
    nblk = N_EDGES // (8 * _BLK)
    return idx.reshape(nblk, 8, _BLK).swapaxes(1, 2).reshape(-1)


def kernel(node_feats, edge_index, edge_feats, W1, b1, W2, b2):
    src = _permute(edge_index[0].astype(jnp.int32))
    dst = _permute(edge_index[1].astype(jnp.int32))
    W1a = W1[:NODE_DIM]
    W1b = W1[NODE_DIM:2 * NODE_DIM]
    W1e = W1[2 * NODE_DIM:]

    P, Q = _project(node_feats, W1a, W1b)
    G1, G2 = _sc_gather(P, Q, src, dst)

    eye8 = jnp.eye(8, dtype=jnp.float32)
    w1e_big = jnp.kron(eye8, W1e)
    w2_big = jnp.kron(eye8, W2)
    b1_big = jnp.tile(b1, 8)[None, :]
    b2_big = jnp.tile(b2, 8)[None, :]

    rows = N_EDGES * EDGE_DIM // 128
    out_t = _edge_mlp(
        G1.reshape(rows, 128), G2.reshape(rows, 128),
        edge_feats.T,
        w1e_big, b1_big, w2_big, b2_big,
    )
    return out_t.T
